# Initial kernel scaffold; baseline (speedup 1.0000x reference)
import jax
import jax.numpy as jnp
from jax.experimental import pallas as pl


def kernel(bitstrings, variables):
    raise NotImplementedError("write your pallas kernel here")


# speedup vs baseline: 5.1559x; 5.1559x over previous
import functools
import itertools

import numpy as np
import jax
import jax.numpy as jnp
from jax import lax
from jax.experimental import pallas as pl
from jax.experimental.pallas import tpu as pltpu
from jax.experimental.pallas import tpu_sc as plsc

_ORDER = 8
_T = 2**_ORDER - 1
_B = 16384
_NC, _NS = 2, 16
_NW = _NC * _NS
_BPW = _B // _NW
_L = 16


def _build_perm():
    combos = []
    for i in range(1, _ORDER + 1):
        combos.extend(itertools.combinations(range(_ORDER), i))
    perm = np.full((256,), _T, dtype=np.int32)
    for t, c in enumerate(combos):
        m = 0
        for j in c:
            m |= 1 << j
        perm[m] = t
    return perm


_PERM = _build_perm()

_MESH = plsc.VectorSubcoreMesh(
    core_axis_name="c", subcore_axis_name="s", num_cores=_NC, num_subcores=_NS
)


@functools.partial(
    pl.kernel,
    mesh=_MESH,
    compiler_params=pltpu.CompilerParams(
        use_tc_tiling_on_sc=False, needs_layout_passes=False
    ),
    out_type=jax.ShapeDtypeStruct((_B,), jnp.float32),
    scratch_types=[
        pltpu.VMEM((_BPW, _ORDER), jnp.int32),
        pltpu.VMEM((256,), jnp.float32),
        pltpu.VMEM((256,), jnp.float32),
        pltpu.VMEM((256,), jnp.int32),
        pltpu.VMEM((_BPW,), jnp.float32),
    ],
)
def _kobe_sc(bits_hbm, varsp_hbm, perm_hbm, out_hbm, bits_v, vars_v, lut_v, perm_v, out_v):
    wid = lax.axis_index("s") * _NC + lax.axis_index("c")
    base = wid * _BPW

    pltpu.sync_copy(bits_hbm.at[pl.ds(base, _BPW), pl.ds(0, _ORDER)], bits_v)
    pltpu.sync_copy(varsp_hbm, vars_v)
    pltpu.sync_copy(perm_hbm, perm_v)

    lane = lax.iota(jnp.int32, _L)

    for i in range(256 // _L):
        idx = perm_v[pl.ds(i * _L, _L)]
        lut_v[pl.ds(i * _L, _L)] = plsc.load_gather(vars_v, [idx])

    for lg in range(4):
        d = 1 << lg
        sign = (1 - 2 * ((lane >> lg) & 1)).astype(jnp.float32)
        pidx = lane ^ d
        for i in range(256 // _L):
            x = lut_v[pl.ds(i * _L, _L)]
            partner = plsc.load_gather(lut_v, [pidx + i * _L])
            lut_v[pl.ds(i * _L, _L)] = partner + sign * x
    for dr in (1, 2, 4, 8):
        for p in range(16):
            if p & dr:
                continue
            a = lut_v[pl.ds(p * _L, _L)]
            b = lut_v[pl.ds((p + dr) * _L, _L)]
            lut_v[pl.ds(p * _L, _L)] = a + b
            lut_v[pl.ds((p + dr) * _L, _L)] = a - b

    for k in range(_BPW // _L):
        rows = lane + k * _L
        acc = plsc.load_gather(bits_v, [rows, jnp.full((_L,), 0, jnp.int32)])
        for j in range(1, _ORDER):
            bj = plsc.load_gather(bits_v, [rows, jnp.full((_L,), j, jnp.int32)])
            acc = acc + (bj << j)
        out_v[pl.ds(k * _L, _L)] = plsc.load_gather(lut_v, [acc])

    pltpu.sync_copy(out_v, out_hbm.at[pl.ds(base, _BPW)])


def kernel(bitstrings, variables):
    varsp = jnp.concatenate(
        [variables.astype(jnp.float32), jnp.zeros((1,), jnp.float32)]
    )
    return _kobe_sc(bitstrings, varsp, jnp.asarray(_PERM))


# speedup vs baseline: 7.6067x; 1.4753x over previous
import functools
import itertools

import numpy as np
import jax
import jax.numpy as jnp
from jax import lax
from jax.experimental import pallas as pl
from jax.experimental.pallas import tpu as pltpu
from jax.experimental.pallas import tpu_sc as plsc

_ORDER = 8
_T = 2**_ORDER - 1
_B = 16384
_NC, _NS = 2, 16
_NW = _NC * _NS
_BPW = _B // _NW
_L = 16


def _build_perm():
    combos = []
    for i in range(1, _ORDER + 1):
        combos.extend(itertools.combinations(range(_ORDER), i))
    perm = np.full((256,), _T, dtype=np.int32)
    for t, c in enumerate(combos):
        m = 0
        for j in c:
            m |= 1 << j
        perm[m] = t
    return perm


_PERM = _build_perm()

_MESH = plsc.VectorSubcoreMesh(
    core_axis_name="c", subcore_axis_name="s", num_cores=_NC, num_subcores=_NS
)


@functools.partial(
    pl.kernel,
    mesh=_MESH,
    compiler_params=pltpu.CompilerParams(needs_layout_passes=False),
    out_type=jax.ShapeDtypeStruct((_B,), jnp.float32),
    scratch_types=[
        pltpu.VMEM((_BPW, 128), jnp.int32),
        pltpu.VMEM((256,), jnp.float32),
        pltpu.VMEM((256,), jnp.float32),
        pltpu.VMEM((256,), jnp.int32),
        pltpu.VMEM((_BPW,), jnp.float32),
    ],
)
def _kobe_sc(bits_hbm, varsp_hbm, perm_hbm, out_hbm, bits_v, vars_v, lut_v, perm_v, out_v):
    wid = lax.axis_index("s") * _NC + lax.axis_index("c")
    base = wid * _BPW

    pltpu.sync_copy(bits_hbm.at[pl.ds(base, _BPW), pl.ds(0, 128)], bits_v)
    pltpu.sync_copy(varsp_hbm, vars_v)
    pltpu.sync_copy(perm_hbm, perm_v)

    lane = lax.iota(jnp.int32, _L)

    for i in range(256 // _L):
        idx = perm_v[pl.ds(i * _L, _L)]
        lut_v[pl.ds(i * _L, _L)] = plsc.load_gather(vars_v, [idx])

    for lg in range(4):
        d = 1 << lg
        sign = (1 - 2 * ((lane >> lg) & 1)).astype(jnp.float32)
        pidx = lane ^ d
        for i in range(256 // _L):
            x = lut_v[pl.ds(i * _L, _L)]
            partner = plsc.load_gather(lut_v, [pidx + i * _L])
            lut_v[pl.ds(i * _L, _L)] = partner + sign * x
    for dr in (1, 2, 4, 8):
        for p in range(16):
            if p & dr:
                continue
            a = lut_v[pl.ds(p * _L, _L)]
            b = lut_v[pl.ds((p + dr) * _L, _L)]
            lut_v[pl.ds(p * _L, _L)] = a + b
            lut_v[pl.ds((p + dr) * _L, _L)] = a - b

    for k in range(_BPW // _L):
        rows = lane + k * _L
        acc = plsc.load_gather(bits_v, [rows, jnp.full((_L,), 0, jnp.int32)])
        for j in range(1, _ORDER):
            bj = plsc.load_gather(bits_v, [rows, jnp.full((_L,), j, jnp.int32)])
            acc = acc + (bj << j)
        out_v[pl.ds(k * _L, _L)] = plsc.load_gather(lut_v, [acc])

    pltpu.sync_copy(out_v, out_hbm.at[pl.ds(base, _BPW)])


def kernel(bitstrings, variables):
    varsp = jnp.concatenate(
        [variables.astype(jnp.float32), jnp.zeros((1,), jnp.float32)]
    )
    return _kobe_sc(bitstrings, varsp, jnp.asarray(_PERM))


# speedup vs baseline: 8.4567x; 1.1117x over previous
import functools
import itertools

import numpy as np
import jax
import jax.numpy as jnp
from jax import lax
from jax.experimental import pallas as pl
from jax.experimental.pallas import tpu as pltpu
from jax.experimental.pallas import tpu_sc as plsc

_ORDER = 8
_T = 2**_ORDER - 1
_B = 16384
_NC, _NS = 2, 16
_NW = _NC * _NS
_BPW = _B // _NW
_L = 16


def _build_perm():
    combos = []
    for i in range(1, _ORDER + 1):
        combos.extend(itertools.combinations(range(_ORDER), i))
    perm = np.zeros((256,), dtype=np.int32)
    for t, c in enumerate(combos):
        m = 0
        for j in c:
            m |= 1 << j
        perm[m] = t
    return perm


_PERM = _build_perm()

_MESH = plsc.VectorSubcoreMesh(
    core_axis_name="c", subcore_axis_name="s", num_cores=_NC, num_subcores=_NS
)


@functools.partial(
    pl.kernel,
    mesh=_MESH,
    compiler_params=pltpu.CompilerParams(needs_layout_passes=False),
    out_type=jax.ShapeDtypeStruct((_B,), jnp.float32),
    scratch_types=[
        pltpu.VMEM((_BPW, 128), jnp.int32),
        pltpu.VMEM((256,), jnp.float32),
        pltpu.VMEM((256,), jnp.float32),
        pltpu.VMEM((256,), jnp.int32),
        pltpu.VMEM((_BPW,), jnp.float32),
        pltpu.SemaphoreType.DMA,
    ],
)
def _kobe_sc(bits_hbm, vars_hbm, perm_hbm, out_hbm, bits_v, vars_v, lut_v, perm_v, out_v, sem):
    wid = lax.axis_index("s") * _NC + lax.axis_index("c")
    base = wid * _BPW

    bits_dma = pltpu.async_copy(
        bits_hbm.at[pl.ds(base, _BPW), pl.ds(0, 128)], bits_v, sem
    )
    pltpu.sync_copy(vars_hbm, vars_v.at[pl.ds(0, _T)])
    pltpu.sync_copy(perm_hbm, perm_v)

    lane = lax.iota(jnp.int32, _L)

    def reorder_body(i, _):
        idx = perm_v[pl.ds(i * _L, _L)]
        lut_v[pl.ds(i * _L, _L)] = plsc.load_gather(vars_v, [idx])
        return _

    lax.fori_loop(0, 256 // _L, reorder_body, 0, unroll=False)
    v0 = lut_v[pl.ds(0, _L)]
    lut_v[pl.ds(0, _L)] = jnp.where(lane == 0, jnp.float32(0.0), v0)

    def inlane_body(it, _):
        lg = it // (256 // _L)
        i = it % (256 // _L)
        d = jnp.int32(1) << lg
        sign = (1 - 2 * ((lane >> lg) & 1)).astype(jnp.float32)
        pidx = (lane ^ d) + i * _L
        x = lut_v[pl.ds(i * _L, _L)]
        partner = plsc.load_gather(lut_v, [pidx])
        lut_v[pl.ds(i * _L, _L)] = partner + sign * x
        return _

    lax.fori_loop(0, 4 * (256 // _L), inlane_body, 0, unroll=False)

    for dr in (1, 2, 4, 8):
        for p in range(16):
            if p & dr:
                continue
            a = lut_v[pl.ds(p * _L, _L)]
            b = lut_v[pl.ds((p + dr) * _L, _L)]
            lut_v[pl.ds(p * _L, _L)] = a + b
            lut_v[pl.ds((p + dr) * _L, _L)] = a - b

    bits_dma.wait()

    def main_body(k, _):
        rows = lane + k * _L
        acc = plsc.load_gather(bits_v, [rows, jnp.zeros((_L,), jnp.int32)])
        for j in range(1, _ORDER):
            bj = plsc.load_gather(bits_v, [rows, jnp.full((_L,), j, jnp.int32)])
            acc = acc + (bj << j)
        out_v[pl.ds(k * _L, _L)] = plsc.load_gather(lut_v, [acc])
        return _

    lax.fori_loop(0, _BPW // _L, main_body, 0, unroll=False)

    pltpu.sync_copy(out_v, out_hbm.at[pl.ds(base, _BPW)])


def kernel(bitstrings, variables):
    return _kobe_sc(bitstrings, variables.astype(jnp.float32), jnp.asarray(_PERM))


# speedup vs baseline: 8.5550x; 1.0116x over previous
import functools
import itertools

import numpy as np
import jax
import jax.numpy as jnp
from jax import lax
from jax.experimental import pallas as pl
from jax.experimental.pallas import tpu as pltpu
from jax.experimental.pallas import tpu_sc as plsc

_ORDER = 8
_T = 2**_ORDER - 1
_B = 16384
_NC, _NS = 2, 16
_NW = _NC * _NS
_BPW = _B // _NW
_L = 16
_CHUNK = _BPW // 2


def _build_perm():
    combos = []
    for i in range(1, _ORDER + 1):
        combos.extend(itertools.combinations(range(_ORDER), i))
    perm = np.zeros((256,), dtype=np.int32)
    for t, c in enumerate(combos):
        m = 0
        for j in c:
            m |= 1 << j
        perm[m] = t
    return perm


_PERM = _build_perm()
_PERM_PACKED = (
    _PERM.reshape(64, 4).astype(np.int64) * (1 << (8 * np.arange(4, dtype=np.int64)))
).sum(axis=1).astype(np.uint32).astype(np.int64)
_PERM_PACKED = [int(x) - (1 << 32) if x >= (1 << 31) else int(x) for x in _PERM_PACKED]

_MESH = plsc.VectorSubcoreMesh(
    core_axis_name="c", subcore_axis_name="s", num_cores=_NC, num_subcores=_NS
)


@functools.partial(
    pl.kernel,
    mesh=_MESH,
    compiler_params=pltpu.CompilerParams(needs_layout_passes=False),
    out_type=jax.ShapeDtypeStruct((_B,), jnp.float32),
    scratch_types=[
        pltpu.VMEM((2, _CHUNK, 128), jnp.int32),
        pltpu.VMEM((256,), jnp.float32),
        pltpu.VMEM((256,), jnp.float32),
        pltpu.VMEM((64,), jnp.int32),
        pltpu.VMEM((_BPW,), jnp.float32),
        pltpu.SemaphoreType.DMA,
        pltpu.SemaphoreType.DMA,
    ],
)
def _kobe_sc(bits_hbm, vars_hbm, out_hbm, bits_v, vars_v, lut_v, packed_v, out_v, sem0, sem1):
    wid = lax.axis_index("s") * _NC + lax.axis_index("c")
    base = wid * _BPW

    dma0 = pltpu.async_copy(
        bits_hbm.at[pl.ds(base, _CHUNK), pl.ds(0, 128)], bits_v.at[0], sem0
    )
    dma1 = pltpu.async_copy(
        bits_hbm.at[pl.ds(base + _CHUNK, _CHUNK), pl.ds(0, 128)], bits_v.at[1], sem1
    )
    pltpu.sync_copy(vars_hbm, vars_v.at[pl.ds(0, _T)])

    lane = lax.iota(jnp.int32, _L)

    for g in range(4):
        acc = jnp.full((_L,), _PERM_PACKED[g * _L], jnp.int32)
        for k in range(1, _L):
            acc = jnp.where(lane == k, jnp.int32(_PERM_PACKED[g * _L + k]), acc)
        packed_v[pl.ds(g * _L, _L)] = acc
    shamt = (lane & 3) * 8
    for i in range(256 // _L):
        pidx = (lane >> 2) + (4 * i)
        pg = plsc.load_gather(packed_v, [pidx])
        idx = (pg >> shamt) & 255
        lut_v[pl.ds(i * _L, _L)] = plsc.load_gather(vars_v, [idx])
    v0 = lut_v[pl.ds(0, _L)]
    lut_v[pl.ds(0, _L)] = jnp.where(lane == 0, jnp.float32(0.0), v0)

    def inlane_body(it, _):
        lg = it // (256 // _L)
        i = it % (256 // _L)
        d = jnp.int32(1) << lg
        sign = (1 - 2 * ((lane >> lg) & 1)).astype(jnp.float32)
        pidx = (lane ^ d) + i * _L
        x = lut_v[pl.ds(i * _L, _L)]
        partner = plsc.load_gather(lut_v, [pidx])
        lut_v[pl.ds(i * _L, _L)] = partner + sign * x
        return _

    lax.fori_loop(0, 4 * (256 // _L), inlane_body, 0, unroll=2)

    for dr in (1, 2, 4, 8):
        for p in range(16):
            if p & dr:
                continue
            a = lut_v[pl.ds(p * _L, _L)]
            b = lut_v[pl.ds((p + dr) * _L, _L)]
            lut_v[pl.ds(p * _L, _L)] = a + b
            lut_v[pl.ds((p + dr) * _L, _L)] = a - b

    def make_body(c):
        def body(k, _):
            rows = lane + k * _L
            acc = plsc.load_gather(
                bits_v, [jnp.full((_L,), c, jnp.int32), rows, jnp.zeros((_L,), jnp.int32)]
            )
            for j in range(1, _ORDER):
                bj = plsc.load_gather(
                    bits_v,
                    [jnp.full((_L,), c, jnp.int32), rows, jnp.full((_L,), j, jnp.int32)],
                )
                acc = acc + (bj << j)
            out_v[pl.ds(c * _CHUNK + k * _L, _L)] = plsc.load_gather(lut_v, [acc])
            return _

        return body

    dma0.wait()
    lax.fori_loop(0, _CHUNK // _L, make_body(0), 0, unroll=2)
    dma1.wait()
    lax.fori_loop(0, _CHUNK // _L, make_body(1), 0, unroll=2)

    pltpu.sync_copy(out_v, out_hbm.at[pl.ds(base, _BPW)])


def kernel(bitstrings, variables):
    return _kobe_sc(bitstrings, variables.astype(jnp.float32))


# speedup vs baseline: 8.5551x; 1.0000x over previous
import functools
import itertools

import numpy as np
import jax
import jax.numpy as jnp
from jax import lax
from jax.experimental import pallas as pl
from jax.experimental.pallas import tpu as pltpu
from jax.experimental.pallas import tpu_sc as plsc

_ORDER = 8
_T = 2**_ORDER - 1
_B = 16384
_NC, _NS = 2, 16
_NW = _NC * _NS
_BPW = _B // _NW
_L = 16
_CHUNK = _BPW // 2


def _build_perm():
    combos = []
    for i in range(1, _ORDER + 1):
        combos.extend(itertools.combinations(range(_ORDER), i))
    perm = np.zeros((256,), dtype=np.int32)
    for t, c in enumerate(combos):
        m = 0
        for j in c:
            m |= 1 << j
        perm[m] = t
    return perm


_PERM = _build_perm()
_PERM_PACKED = (
    _PERM.reshape(64, 4).astype(np.int64) * (1 << (8 * np.arange(4, dtype=np.int64)))
).sum(axis=1).astype(np.uint32).astype(np.int64)
_PERM_PACKED = [int(x) - (1 << 32) if x >= (1 << 31) else int(x) for x in _PERM_PACKED]

_MESH = plsc.VectorSubcoreMesh(
    core_axis_name="c", subcore_axis_name="s", num_cores=_NC, num_subcores=_NS
)


@functools.partial(
    pl.kernel,
    mesh=_MESH,
    compiler_params=pltpu.CompilerParams(
        needs_layout_passes=False, skip_device_barrier=True
    ),
    out_type=jax.ShapeDtypeStruct((_B,), jnp.float32),
    scratch_types=[
        pltpu.VMEM((2, _CHUNK, 128), jnp.int32),
        pltpu.VMEM((256,), jnp.float32),
        pltpu.VMEM((256,), jnp.float32),
        pltpu.VMEM((64,), jnp.int32),
        pltpu.VMEM((_BPW,), jnp.float32),
        pltpu.SemaphoreType.DMA,
        pltpu.SemaphoreType.DMA,
    ],
)
def _kobe_sc(bits_hbm, vars_hbm, out_hbm, bits_v, vars_v, lut_v, packed_v, out_v, sem0, sem1):
    wid = lax.axis_index("s") * _NC + lax.axis_index("c")
    base = wid * _BPW

    dma0 = pltpu.async_copy(
        bits_hbm.at[pl.ds(base, _CHUNK), pl.ds(0, 128)], bits_v.at[0], sem0
    )
    dma1 = pltpu.async_copy(
        bits_hbm.at[pl.ds(base + _CHUNK, _CHUNK), pl.ds(0, 128)], bits_v.at[1], sem1
    )
    pltpu.sync_copy(vars_hbm, vars_v.at[pl.ds(0, _T)])

    lane = lax.iota(jnp.int32, _L)

    for g in range(4):
        acc = jnp.full((_L,), _PERM_PACKED[g * _L], jnp.int32)
        for k in range(1, _L):
            acc = jnp.where(lane == k, jnp.int32(_PERM_PACKED[g * _L + k]), acc)
        packed_v[pl.ds(g * _L, _L)] = acc
    shamt = (lane & 3) * 8
    for i in range(256 // _L):
        pidx = (lane >> 2) + (4 * i)
        pg = plsc.load_gather(packed_v, [pidx])
        idx = (pg >> shamt) & 255
        lut_v[pl.ds(i * _L, _L)] = plsc.load_gather(vars_v, [idx])
    v0 = lut_v[pl.ds(0, _L)]
    lut_v[pl.ds(0, _L)] = jnp.where(lane == 0, jnp.float32(0.0), v0)

    def inlane_body(it, _):
        lg = it // (256 // _L)
        i = it % (256 // _L)
        d = jnp.int32(1) << lg
        sign = (1 - 2 * ((lane >> lg) & 1)).astype(jnp.float32)
        pidx = (lane ^ d) + i * _L
        x = lut_v[pl.ds(i * _L, _L)]
        partner = plsc.load_gather(lut_v, [pidx])
        lut_v[pl.ds(i * _L, _L)] = partner + sign * x
        return _

    lax.fori_loop(0, 4 * (256 // _L), inlane_body, 0, unroll=2)

    for dr in (1, 2, 4, 8):
        for p in range(16):
            if p & dr:
                continue
            a = lut_v[pl.ds(p * _L, _L)]
            b = lut_v[pl.ds((p + dr) * _L, _L)]
            lut_v[pl.ds(p * _L, _L)] = a + b
            lut_v[pl.ds((p + dr) * _L, _L)] = a - b

    def make_body(c):
        def body(k, _):
            rows = lane + k * _L
            acc = plsc.load_gather(
                bits_v, [jnp.full((_L,), c, jnp.int32), rows, jnp.zeros((_L,), jnp.int32)]
            )
            for j in range(1, _ORDER):
                bj = plsc.load_gather(
                    bits_v,
                    [jnp.full((_L,), c, jnp.int32), rows, jnp.full((_L,), j, jnp.int32)],
                )
                acc = acc + (bj << j)
            out_v[pl.ds(c * _CHUNK + k * _L, _L)] = plsc.load_gather(lut_v, [acc])
            return _

        return body

    dma0.wait()
    lax.fori_loop(0, _CHUNK // _L, make_body(0), 0, unroll=2)
    dma1.wait()
    lax.fori_loop(0, _CHUNK // _L, make_body(1), 0, unroll=2)

    pltpu.sync_copy(out_v, out_hbm.at[pl.ds(base, _BPW)])


def kernel(bitstrings, variables):
    return _kobe_sc(bitstrings, variables.astype(jnp.float32))
